# baseline (speedup 1.0000x reference)
import functools

import jax
import jax.numpy as jnp
from jax import lax
from jax.experimental import pallas as pl
from jax.experimental.pallas import tpu as pltpu
from jax.experimental.pallas import tpu_sc as plsc

_N = 50000
_E = 800000
_D = 128
_G = 64
_H = 64
_HALF = 32

_NC = 2
_NS = 16
_CHUNK = 128
_SB = 20
_NBUF = 5
_CPT = 400
_NSB = _CPT // _SB
_EPT = _CPT * _CHUNK
_EPAD = _NS * _EPT


_NP = 50048
_RPT8 = _NP // _NS

_BLK = 2000
_NBLK = _N // _BLK

_BN_S = 1.0 / (1.0 + 1e-5) ** 0.5


@functools.cache
def _make_sc_gin_agg():
    mesh = plsc.VectorSubcoreMesh(
        core_axis_name="c", subcore_axis_name="s",
        num_cores=_NC, num_subcores=_NS)
    return pl.kernel(
        _sc_gin_agg_body,
        out_type=jax.ShapeDtypeStruct((_NC, _NP, _HALF), jnp.float32),
        mesh=mesh,
        scratch_types=[
            pltpu.VMEM((2, _SB, 2, _CHUNK), jnp.int32),
            pltpu.VMEM((_NBUF, _CHUNK, _HALF), jnp.float32),
            pltpu.SemaphoreType.DMA,
            pltpu.SemaphoreType.DMA,
            pltpu.SemaphoreType.DMA,
            pltpu.SemaphoreType.DMA,
            pltpu.SemaphoreType.DMA,
            pltpu.SemaphoreType.DMA,
            pltpu.VMEM_SHARED((_NP + 8, _HALF), jnp.float32),
        ],
        compiler_params=pltpu.CompilerParams(use_tc_tiling_on_sc=False),
    )


def _sc_gin_agg(htab, srcdst):
    return _make_sc_gin_agg()(htab, srcdst)


def _sc_gin_agg_body(htab, srcdst, out, idx2, rows, sidx,
                     sr0, sr1, sr2, sr3, sr4, acc):
    c = lax.axis_index("c")
    s = lax.axis_index("s")
    rsems = (sr0, sr1, sr2, sr3, sr4)
    base = s * _CPT

    pltpu.sync_copy(htab.at[pl.ds(c * _NP + s * _RPT8, _RPT8)],
                    acc.at[pl.ds(s * _RPT8, _RPT8)])
    plsc.subcore_barrier()

    pltpu.sync_copy(srcdst.at[c, pl.ds(base, _SB)], idx2.at[0])
    for b in range(_NBUF):
        pltpu.async_copy(htab.at[idx2.at[0, b, 0]], rows.at[b], rsems[b])

    def sb_body(sb, carry):
        cur = lax.rem(sb, 2)
        nxt = 1 - cur

        @pl.when(sb < _NSB - 1)
        def _():
            pltpu.async_copy(srcdst.at[c, pl.ds(base + (sb + 1) * _SB, _SB)],
                             idx2.at[nxt], sidx)

        for j in range(_SB // _NBUF):
            if j == _SB // _NBUF - 1:
                @pl.when(sb < _NSB - 1)
                def _():
                    pltpu.make_async_copy(srcdst.at[c, pl.ds(base, _SB)],
                                          idx2.at[nxt], sidx).wait()
            for b in range(_NBUF):
                i = j * _NBUF + b
                pltpu.make_async_copy(htab.at[idx2.at[cur, i, 0]],
                                      rows.at[b], rsems[b]).wait()
                pltpu.sync_copy(rows.at[b], acc.at[idx2.at[cur, i, 1]],
                                add=True)
                if i + _NBUF < _SB:
                    pltpu.async_copy(htab.at[idx2.at[cur, i + _NBUF, 0]],
                                     rows.at[b], rsems[b])
                else:
                    @pl.when(sb < _NSB - 1)
                    def _(b=b, i=i):
                        pltpu.async_copy(
                            htab.at[idx2.at[nxt, i + _NBUF - _SB, 0]],
                            rows.at[b], rsems[b])
        return carry

    lax.fori_loop(0, _NSB, sb_body, 0)
    plsc.subcore_barrier()
    pltpu.sync_copy(acc.at[pl.ds(s * _RPT8, _RPT8)],
                    out.at[c, pl.ds(s * _RPT8, _RPT8)])


def _full(shape):
    return pl.BlockSpec(shape, lambda i: tuple(0 for _ in shape))


def _pre_body(x_ref, w1, b1, w2, b2, w3, b3, w4, b4, newx_ref, h2_ref):
    xb = x_ref[...]
    hs = jnp.maximum(jnp.dot(xb, w1[...], preferred_element_type=jnp.float32)
                     + b1[...], 0.0)
    hs = jnp.maximum(jnp.dot(hs, w2[...], preferred_element_type=jnp.float32)
                     + b2[...], 0.0)
    hi = jnp.maximum(jnp.dot(xb, w3[...], preferred_element_type=jnp.float32)
                     + b3[...], 0.0)
    hi = jnp.maximum(jnp.dot(hi, w4[...], preferred_element_type=jnp.float32)
                     + b4[...], 0.0)
    newx_ref[...] = jnp.concatenate([hi, hs], axis=1)
    h2_ref[0] = hi
    h2_ref[1] = hs


def _gin_mlp0_body(z2_ref, newx_ref, wa, ba, wb, bb, sc, btc, sbn, btbn,
                   g0_ref, h2_ref):
    zb = jnp.concatenate([z2_ref[0], z2_ref[1]], axis=1)
    t = jnp.dot(zb, wa[...], preferred_element_type=jnp.float32) + ba[...]
    t = jnp.maximum(t * sc[...] + btc[...], 0.0)
    u = jnp.dot(t, wb[...], preferred_element_type=jnp.float32) + bb[...]
    g0 = u * sbn[...] + btbn[...] + newx_ref[...]
    g0_ref[...] = g0
    h2_ref[0] = g0[:, :_HALF]
    h2_ref[1] = g0[:, _HALF:]


def _gin_mlp1_body(z2_ref, g0_ref, newx_ref, batch_ref, wa, ba, wb, bb,
                   sc, btc, sbn, btbn, sums_ref, cnt_ref):
    i = pl.program_id(0)
    zb = jnp.concatenate([z2_ref[0], z2_ref[1]], axis=1)
    t = jnp.dot(zb, wa[...], preferred_element_type=jnp.float32) + ba[...]
    t = jnp.maximum(t * sc[...] + btc[...], 0.0)
    u = jnp.dot(t, wb[...], preferred_element_type=jnp.float32) + bb[...]
    g1 = u * sbn[...] + btbn[...] + g0_ref[...] + newx_ref[...]
    bids = batch_ref[0, 0, :]
    oh = (bids[None, :] == lax.broadcasted_iota(jnp.int32, (_G, _BLK), 0)
          ).astype(jnp.float32)
    part_sums = jnp.dot(oh, g1, preferred_element_type=jnp.float32)
    part_cnt = jnp.broadcast_to(jnp.sum(oh, axis=1, keepdims=True), (_G, _H))

    @pl.when(i == 0)
    def _():
        sums_ref[...] = jnp.zeros_like(sums_ref)
        cnt_ref[...] = jnp.zeros_like(cnt_ref)

    sums_ref[...] += part_sums
    cnt_ref[...] += part_cnt


def _post_body(sums_ref, cnt_ref, w1, b1, w2, b2, out_ref):
    pooled = sums_ref[...] / jnp.maximum(cnt_ref[...], 1.0)
    h = jnp.maximum(jnp.dot(pooled, w1[...], preferred_element_type=jnp.float32)
                    + b1[...], 0.0)
    logits = jnp.dot(h, w2[...], preferred_element_type=jnp.float32) + b2[...]
    m = jnp.max(logits, axis=1, keepdims=True)
    z = logits - m
    out_ref[...] = z - jnp.log(jnp.sum(jnp.exp(z), axis=1, keepdims=True))


def kernel(x, edge_index, batch, params):
    p = params
    f32 = jnp.float32

    w1t = jnp.zeros((_D, 16), f32).at[_D - 2:, :].set(p['w_pre1'].T)
    w3t = jnp.zeros((_D, 16), f32).at[:_D - 2, :].set(p['w_pre3'].T)
    row = lambda v: v.reshape(1, -1)
    pre_ws = (w1t, row(p['b_pre1']), p['w_pre2'].T, row(p['b_pre2']),
              w3t, row(p['b_pre3']), p['w_pre4'].T, row(p['b_pre4']))

    def mlp_ws(i):
        return (p['w_c%da' % i].T, row(p['b_c%da' % i]),
                p['w_c%db' % i].T, row(p['b_c%db' % i]),
                row(p['g_c%d' % i]) * _BN_S, row(p['bt_c%d' % i]),
                row(p['g_bn%d' % i]) * _BN_S, row(p['bt_bn%d' % i]))

    src = edge_index[0]
    dst = edge_index[1]
    npad = _EPAD - _E
    srcp = jnp.concatenate([src, jnp.zeros((npad,), jnp.int32)]
                           ).reshape(_NS * _CPT, 1, _CHUNK)
    dstp = jnp.concatenate([dst, jnp.full((npad,), _NP, jnp.int32)]
                           ).reshape(_NS * _CPT, 1, _CHUNK)
    srcdst = jnp.stack([
        jnp.concatenate([srcp, dstp], axis=1),
        jnp.concatenate([srcp + _NP, dstp], axis=1)])
    batch3 = batch.reshape(_NBLK, 1, _BLK)

    row_spec = pl.BlockSpec((_BLK, _D), lambda i: (i, 0))
    h64_spec = pl.BlockSpec((_BLK, _H), lambda i: (i, 0))
    h2_spec = pl.BlockSpec((_NC, _BLK, _HALF), lambda i: (0, i, 0))
    w_specs = lambda ws: [_full(w.shape) for w in ws]

    new_x, h2 = pl.pallas_call(
        _pre_body,
        grid=(_NBLK,),
        in_specs=[row_spec] + w_specs(pre_ws),
        out_specs=[h64_spec, h2_spec],
        out_shape=[jax.ShapeDtypeStruct((_N, _H), f32),
                   jax.ShapeDtypeStruct((_NC, _NP, _HALF), f32)],
    )(x, *pre_ws)

    z0_2 = _sc_gin_agg(h2.reshape(_NC * _NP, _HALF), srcdst)

    ws0 = mlp_ws(0)
    g0, g0h2 = pl.pallas_call(
        _gin_mlp0_body,
        grid=(_NBLK,),
        in_specs=[h2_spec, h64_spec] + w_specs(ws0),
        out_specs=[h64_spec, h2_spec],
        out_shape=[jax.ShapeDtypeStruct((_N, _H), f32),
                   jax.ShapeDtypeStruct((_NC, _NP, _HALF), f32)],
    )(z0_2, new_x, *ws0)

    z1_2 = _sc_gin_agg(g0h2.reshape(_NC * _NP, _HALF), srcdst)

    ws1 = mlp_ws(1)
    acc_spec = pl.BlockSpec((_G, _H), lambda i: (0, 0))
    batch_spec = pl.BlockSpec((1, 1, _BLK), lambda i: (i, 0, 0))
    sums, cnt = pl.pallas_call(
        _gin_mlp1_body,
        grid=(_NBLK,),
        in_specs=[h2_spec, h64_spec, h64_spec, batch_spec] + w_specs(ws1),
        out_specs=[acc_spec, acc_spec],
        out_shape=[jax.ShapeDtypeStruct((_G, _H), f32),
                   jax.ShapeDtypeStruct((_G, _H), f32)],
    )(z1_2, g0, new_x, batch3, *ws1)

    post_ws = (p['w_post1'].T, row(p['b_post1']),
               p['w_post2'].T, row(p['b_post2']))
    out = pl.pallas_call(
        _post_body,
        out_shape=jax.ShapeDtypeStruct((_G, 7), f32),
    )(sums, cnt, *post_ws)
    return out


# speedup vs baseline: 1.5595x; 1.5595x over previous
import functools

import jax
import jax.numpy as jnp
from jax import lax
from jax.experimental import pallas as pl
from jax.experimental.pallas import tpu as pltpu
from jax.experimental.pallas import tpu_sc as plsc

_N = 50000
_E = 800000
_D = 128
_G = 64
_H = 64
_HALF = 32

_NC = 2
_NS = 16
_CHUNK = 128
_SB = 28
_NBUF = 4
_CPT = 392
_NSB = _CPT // _SB
_EPT = _CPT * _CHUNK
_EPAD = _NS * _EPT


_NP = 50048
_RPT8 = _NP // _NS

_BLK = 2000
_NBLK = _N // _BLK

_BN_S = 1.0 / (1.0 + 1e-5) ** 0.5


@functools.cache
def _make_sc_gin_agg():
    mesh = plsc.VectorSubcoreMesh(
        core_axis_name="c", subcore_axis_name="s",
        num_cores=_NC, num_subcores=_NS)
    return pl.kernel(
        _sc_gin_agg_body,
        out_type=jax.ShapeDtypeStruct((_NC, _NP, _HALF), jnp.float32),
        mesh=mesh,
        scratch_types=[
            pltpu.VMEM((2, _SB, 2, _CHUNK), jnp.int32),
            pltpu.VMEM((_NBUF, _CHUNK, _HALF), jnp.float32),
            pltpu.SemaphoreType.DMA,
            pltpu.SemaphoreType.DMA,
            pltpu.SemaphoreType.DMA,
            pltpu.SemaphoreType.DMA,
            pltpu.SemaphoreType.DMA,
            pltpu.VMEM_SHARED((_NP + 8, _HALF), jnp.float32),
        ],
        compiler_params=pltpu.CompilerParams(use_tc_tiling_on_sc=False),
    )


def _sc_gin_agg(htab, srcdst):
    return _make_sc_gin_agg()(htab, srcdst)


def _sc_gin_agg_body(htab, srcdst, out, idx2, rows, sidx,
                     sr0, sr1, sr2, sr3, acc):
    c = lax.axis_index("c")
    s = lax.axis_index("s")
    rsems = (sr0, sr1, sr2, sr3)
    base = s * _CPT

    pltpu.sync_copy(htab.at[pl.ds(c * _NP + s * _RPT8, _RPT8)],
                    acc.at[pl.ds(s * _RPT8, _RPT8)])
    plsc.subcore_barrier()

    pltpu.sync_copy(srcdst.at[c, pl.ds(base, _SB)], idx2.at[0])
    for b in range(_NBUF):
        pltpu.async_copy(htab.at[idx2.at[0, b, 0]], rows.at[b], rsems[b])

    def sb_body(sb, carry):
        cur = lax.rem(sb, 2)
        nxt = 1 - cur

        @pl.when(sb < _NSB - 1)
        def _():
            pltpu.async_copy(srcdst.at[c, pl.ds(base + (sb + 1) * _SB, _SB)],
                             idx2.at[nxt], sidx)

        for j in range(_SB // _NBUF):
            if j == _SB // _NBUF - 1:
                @pl.when(sb < _NSB - 1)
                def _():
                    pltpu.make_async_copy(srcdst.at[c, pl.ds(base, _SB)],
                                          idx2.at[nxt], sidx).wait()
            for b in range(_NBUF):
                i = j * _NBUF + b
                pltpu.make_async_copy(htab.at[idx2.at[cur, i, 0]],
                                      rows.at[b], rsems[b]).wait()
                pltpu.sync_copy(rows.at[b], acc.at[idx2.at[cur, i, 1]],
                                add=True)
                if i + _NBUF < _SB:
                    pltpu.async_copy(htab.at[idx2.at[cur, i + _NBUF, 0]],
                                     rows.at[b], rsems[b])
                else:
                    @pl.when(sb < _NSB - 1)
                    def _(b=b, i=i):
                        pltpu.async_copy(
                            htab.at[idx2.at[nxt, i + _NBUF - _SB, 0]],
                            rows.at[b], rsems[b])
        return carry

    lax.fori_loop(0, _NSB, sb_body, 0)
    plsc.subcore_barrier()
    pltpu.sync_copy(acc.at[pl.ds(s * _RPT8, _RPT8)],
                    out.at[c, pl.ds(s * _RPT8, _RPT8)])


def _full(shape):
    return pl.BlockSpec(shape, lambda i: tuple(0 for _ in shape))


def _pre_body(x_ref, w1, b1, w2, b2, w3, b3, w4, b4, newx_ref, h2_ref):
    xb = x_ref[...]
    hs = jnp.maximum(jnp.dot(xb, w1[...], preferred_element_type=jnp.float32)
                     + b1[...], 0.0)
    hs = jnp.maximum(jnp.dot(hs, w2[...], preferred_element_type=jnp.float32)
                     + b2[...], 0.0)
    hi = jnp.maximum(jnp.dot(xb, w3[...], preferred_element_type=jnp.float32)
                     + b3[...], 0.0)
    hi = jnp.maximum(jnp.dot(hi, w4[...], preferred_element_type=jnp.float32)
                     + b4[...], 0.0)
    newx_ref[...] = jnp.concatenate([hi, hs], axis=1)
    h2_ref[0] = hi
    h2_ref[1] = hs


def _gin_mlp0_body(z2_ref, newx_ref, wa, ba, wb, bb, sc, btc, sbn, btbn,
                   g0_ref, h2_ref):
    zb = jnp.concatenate([z2_ref[0], z2_ref[1]], axis=1)
    t = jnp.dot(zb, wa[...], preferred_element_type=jnp.float32) + ba[...]
    t = jnp.maximum(t * sc[...] + btc[...], 0.0)
    u = jnp.dot(t, wb[...], preferred_element_type=jnp.float32) + bb[...]
    g0 = u * sbn[...] + btbn[...] + newx_ref[...]
    g0_ref[...] = g0
    h2_ref[0] = g0[:, :_HALF]
    h2_ref[1] = g0[:, _HALF:]


def _gin_mlp1_body(z2_ref, g0_ref, newx_ref, batch_ref, wa, ba, wb, bb,
                   sc, btc, sbn, btbn, sums_ref, cnt_ref):
    i = pl.program_id(0)
    zb = jnp.concatenate([z2_ref[0], z2_ref[1]], axis=1)
    t = jnp.dot(zb, wa[...], preferred_element_type=jnp.float32) + ba[...]
    t = jnp.maximum(t * sc[...] + btc[...], 0.0)
    u = jnp.dot(t, wb[...], preferred_element_type=jnp.float32) + bb[...]
    g1 = u * sbn[...] + btbn[...] + g0_ref[...] + newx_ref[...]
    bids = batch_ref[0, 0, :]
    oh = (bids[None, :] == lax.broadcasted_iota(jnp.int32, (_G, _BLK), 0)
          ).astype(jnp.float32)
    part_sums = jnp.dot(oh, g1, preferred_element_type=jnp.float32)
    part_cnt = jnp.broadcast_to(jnp.sum(oh, axis=1, keepdims=True), (_G, _H))

    @pl.when(i == 0)
    def _():
        sums_ref[...] = jnp.zeros_like(sums_ref)
        cnt_ref[...] = jnp.zeros_like(cnt_ref)

    sums_ref[...] += part_sums
    cnt_ref[...] += part_cnt


def _post_body(sums_ref, cnt_ref, w1, b1, w2, b2, out_ref):
    pooled = sums_ref[...] / jnp.maximum(cnt_ref[...], 1.0)
    h = jnp.maximum(jnp.dot(pooled, w1[...], preferred_element_type=jnp.float32)
                    + b1[...], 0.0)
    logits = jnp.dot(h, w2[...], preferred_element_type=jnp.float32) + b2[...]
    m = jnp.max(logits, axis=1, keepdims=True)
    z = logits - m
    out_ref[...] = z - jnp.log(jnp.sum(jnp.exp(z), axis=1, keepdims=True))


def kernel(x, edge_index, batch, params):
    p = params
    f32 = jnp.float32

    w1t = jnp.zeros((_D, 16), f32).at[_D - 2:, :].set(p['w_pre1'].T)
    w3t = jnp.zeros((_D, 16), f32).at[:_D - 2, :].set(p['w_pre3'].T)
    row = lambda v: v.reshape(1, -1)
    pre_ws = (w1t, row(p['b_pre1']), p['w_pre2'].T, row(p['b_pre2']),
              w3t, row(p['b_pre3']), p['w_pre4'].T, row(p['b_pre4']))

    def mlp_ws(i):
        return (p['w_c%da' % i].T, row(p['b_c%da' % i]),
                p['w_c%db' % i].T, row(p['b_c%db' % i]),
                row(p['g_c%d' % i]) * _BN_S, row(p['bt_c%d' % i]),
                row(p['g_bn%d' % i]) * _BN_S, row(p['bt_bn%d' % i]))

    src = edge_index[0]
    dst = edge_index[1]
    npad = _EPAD - _E
    srcp = jnp.concatenate([src, jnp.zeros((npad,), jnp.int32)]
                           ).reshape(_NS * _CPT, 1, _CHUNK)
    dstp = jnp.concatenate([dst, jnp.full((npad,), _NP, jnp.int32)]
                           ).reshape(_NS * _CPT, 1, _CHUNK)
    srcdst = jnp.stack([
        jnp.concatenate([srcp, dstp], axis=1),
        jnp.concatenate([srcp + _NP, dstp], axis=1)])
    batch3 = batch.reshape(_NBLK, 1, _BLK)

    row_spec = pl.BlockSpec((_BLK, _D), lambda i: (i, 0))
    h64_spec = pl.BlockSpec((_BLK, _H), lambda i: (i, 0))
    h2_spec = pl.BlockSpec((_NC, _BLK, _HALF), lambda i: (0, i, 0))
    w_specs = lambda ws: [_full(w.shape) for w in ws]

    new_x, h2 = pl.pallas_call(
        _pre_body,
        grid=(_NBLK,),
        in_specs=[row_spec] + w_specs(pre_ws),
        out_specs=[h64_spec, h2_spec],
        out_shape=[jax.ShapeDtypeStruct((_N, _H), f32),
                   jax.ShapeDtypeStruct((_NC, _NP, _HALF), f32)],
    )(x, *pre_ws)

    z0_2 = _sc_gin_agg(h2.reshape(_NC * _NP, _HALF), srcdst)

    ws0 = mlp_ws(0)
    g0, g0h2 = pl.pallas_call(
        _gin_mlp0_body,
        grid=(_NBLK,),
        in_specs=[h2_spec, h64_spec] + w_specs(ws0),
        out_specs=[h64_spec, h2_spec],
        out_shape=[jax.ShapeDtypeStruct((_N, _H), f32),
                   jax.ShapeDtypeStruct((_NC, _NP, _HALF), f32)],
    )(z0_2, new_x, *ws0)

    z1_2 = _sc_gin_agg(g0h2.reshape(_NC * _NP, _HALF), srcdst)

    ws1 = mlp_ws(1)
    acc_spec = pl.BlockSpec((_G, _H), lambda i: (0, 0))
    batch_spec = pl.BlockSpec((1, 1, _BLK), lambda i: (i, 0, 0))
    sums, cnt = pl.pallas_call(
        _gin_mlp1_body,
        grid=(_NBLK,),
        in_specs=[h2_spec, h64_spec, h64_spec, batch_spec] + w_specs(ws1),
        out_specs=[acc_spec, acc_spec],
        out_shape=[jax.ShapeDtypeStruct((_G, _H), f32),
                   jax.ShapeDtypeStruct((_G, _H), f32)],
    )(z1_2, g0, new_x, batch3, *ws1)

    post_ws = (p['w_post1'].T, row(p['b_post1']),
               p['w_post2'].T, row(p['b_post2']))
    out = pl.pallas_call(
        _post_body,
        out_shape=jax.ShapeDtypeStruct((_G, 7), f32),
    )(sums, cnt, *post_ws)
    return out
